# baseline (speedup 1.0000x reference)
import functools

import numpy as np
import jax
import jax.numpy as jnp
from jax.experimental import pallas as pl
from jax.experimental.pallas import tpu as pltpu

N = 10000
E = 160000
D_IN = 128
MUL = 16
RB = 8
RH = 32
POOL_H = 128
OUT_DIM = 512
NB = 16

TE = 2000

_CENTERS = np.linspace(0.0, 6.0, RB, dtype=np.float32).reshape(1, RB)
_RBF_NORM = np.float32(1.0 / (np.sqrt(2.0 * 3.14159) * 0.5))


def _silu(v):
    return v * jax.nn.sigmoid(v)


def _edge_body(rel_ref, hsrc_ref, w1_ref, b1_ref, w2_ref, b2_ref,
               w3_ref, b3_ref, msg_ref):
    rel = rel_ref[...]
    d2 = jnp.sum(rel * rel, axis=1, keepdims=True)
    dist = jnp.maximum(jnp.sqrt(d2), 1e-6)
    centers = (jax.lax.broadcasted_iota(jnp.int32, (1, RB), 1)
               .astype(jnp.float32) * np.float32(6.0 / (RB - 1)))
    rbf = jnp.exp(-((dist - centers) ** 2) * 2.0) * _RBF_NORM
    hh = _silu(jnp.dot(rbf, w1_ref[...], preferred_element_type=jnp.float32)
               + b1_ref[...])
    hh = _silu(jnp.dot(hh, w2_ref[...], preferred_element_type=jnp.float32)
               + b2_ref[...])
    tw = (jnp.dot(hh, w3_ref[...], preferred_element_type=jnp.float32)
          + b3_ref[...])
    hsrc = hsrc_ref[...]
    acc = tw[:, 0:MUL] * hsrc[:, 0:1]
    for u in range(1, MUL):
        acc = acc + tw[:, u * MUL:(u + 1) * MUL] * hsrc[:, u:u + 1]
    msg_ref[...] = acc * np.float32(1.0 / np.sqrt(MUL))


def _edge_msg(rel, hsrc, w1, b1, w2, b2, w3, b3):
    grid = (E // TE,)
    return pl.pallas_call(
        _edge_body,
        grid=grid,
        in_specs=[
            pl.BlockSpec((TE, 3), lambda i: (i, 0)),
            pl.BlockSpec((TE, MUL), lambda i: (i, 0)),
            pl.BlockSpec((RB, RH), lambda i: (0, 0)),
            pl.BlockSpec((1, RH), lambda i: (0, 0)),
            pl.BlockSpec((RH, RH), lambda i: (0, 0)),
            pl.BlockSpec((1, RH), lambda i: (0, 0)),
            pl.BlockSpec((RH, MUL * MUL), lambda i: (0, 0)),
            pl.BlockSpec((1, MUL * MUL), lambda i: (0, 0)),
        ],
        out_specs=pl.BlockSpec((TE, MUL), lambda i: (i, 0)),
        out_shape=jax.ShapeDtypeStruct((E, MUL), jnp.float32),
    )(rel, hsrc, w1, b1.reshape(1, RH), w2, b2.reshape(1, RH),
      w3, b3.reshape(1, MUL * MUL))


def _matmul_body(x_ref, w_ref, o_ref):
    o_ref[...] = jnp.dot(x_ref[...], w_ref[...],
                         preferred_element_type=jnp.float32)


def _matmul(x, w):
    m, k = x.shape
    _, n = w.shape
    return pl.pallas_call(
        _matmul_body,
        out_shape=jax.ShapeDtypeStruct((m, n), jnp.float32),
    )(x, w)


def _affine_add_body(h_ref, w_ref, agg_ref, o_ref):
    o_ref[...] = (jnp.dot(h_ref[...], w_ref[...],
                          preferred_element_type=jnp.float32)
                  + agg_ref[...])


def _self_plus_agg(h, w, agg):
    return pl.pallas_call(
        _affine_add_body,
        out_shape=jax.ShapeDtypeStruct(h.shape, jnp.float32),
    )(h, w, agg)


def _tail_body(h_ref, batch_ref, p1w_ref, p1b_ref, p2w_ref, p2b_ref,
               p3w_ref, p3b_ref, ow_ref, ob_ref, lng_ref, lnb_ref, out_ref):
    h = h_ref[...]
    a = _silu(jnp.dot(h, p1w_ref[...], preferred_element_type=jnp.float32)
              + p1b_ref[...])
    a = _silu(jnp.dot(a, p2w_ref[...], preferred_element_type=jnp.float32)
              + p2b_ref[...])
    logits = (jnp.dot(a, p3w_ref[...], preferred_element_type=jnp.float32)
              + p3b_ref[...])
    batch = batch_ref[...]
    seg_ids = jax.lax.broadcasted_iota(jnp.int32, (N, NB), 1)
    onehot = batch == seg_ids
    onehot_f = onehot.astype(jnp.float32)
    masked = jnp.where(onehot, logits, jnp.float32(-3e38))
    seg_max = jnp.max(masked, axis=0, keepdims=True)
    maxn = jnp.sum(onehot_f * seg_max, axis=1, keepdims=True)
    zexp = jnp.exp(logits - maxn)
    denom = jnp.sum(onehot_f * zexp, axis=0, keepdims=True)
    wgt = onehot_f * zexp
    g = jax.lax.dot_general(wgt, h, (((0,), (0,)), ((), ())),
                            preferred_element_type=jnp.float32)
    g = g / denom.reshape(NB, 1)
    o = (jnp.dot(g, ow_ref[...], preferred_element_type=jnp.float32)
         + ob_ref[...])
    mu = jnp.mean(o, axis=1, keepdims=True)
    var = jnp.mean(o * o, axis=1, keepdims=True) - mu * mu
    out_ref[...] = ((o - mu) * jax.lax.rsqrt(var + 1e-5)
                    * lng_ref[...] + lnb_ref[...])


def _tail(h, batch, params):
    return pl.pallas_call(
        _tail_body,
        out_shape=jax.ShapeDtypeStruct((NB, OUT_DIM), jnp.float32),
    )(h, batch.reshape(N, 1),
      params['p1_w'], params['p1_b'].reshape(1, POOL_H),
      params['p2_w'], params['p2_b'].reshape(1, POOL_H),
      params['p3_w'], params['p3_b'].reshape(1, 1),
      params['o_w'], params['o_b'].reshape(1, OUT_DIM),
      params['ln_g'].reshape(1, OUT_DIM), params['ln_b'].reshape(1, OUT_DIM))


def kernel(x, pos, edge_index, batch, params):
    src = edge_index[0]
    dst = edge_index[1]
    h = _matmul(x, params['W_in'] / np.sqrt(D_IN))
    rel = pos[dst] - pos[src]
    for lp in params['layers']:
        hsrc = h[src]
        msg = _edge_msg(rel, hsrc, lp['w1'], lp['b1'], lp['w2'], lp['b2'],
                        lp['w3'], lp['b3'])
        agg = jax.ops.segment_sum(msg, dst, num_segments=N)
        h = _self_plus_agg(h, lp['W_si'] / np.sqrt(MUL), agg)
    return _tail(h, batch, params)


# speedup vs baseline: 1.9431x; 1.9431x over previous
import functools

import numpy as np
import jax
import jax.numpy as jnp
from jax import lax
from jax.experimental import pallas as pl
from jax.experimental.pallas import tpu as pltpu
from jax.experimental.pallas import tpu_sc as plsc

N = 10000
E = 160000
D_IN = 128
MUL = 16
RB = 8
RH = 32
POOL_H = 128
OUT_DIM = 512
NB = 16

TE = 2000
NW = 32
EPW = E // NW
NPT = N // 16

_SC_MESH = plsc.VectorSubcoreMesh(core_axis_name="c", subcore_axis_name="s")
_SC_PARAMS = pltpu.CompilerParams(use_tc_tiling_on_sc=False)
_RBF_NORM = np.float32(1.0 / (np.sqrt(2.0 * 3.14159) * 0.5))


def _silu(v):
    return v * jax.nn.sigmoid(v)


def _sc_gather(table, idx):
    @functools.partial(
        pl.kernel,
        out_type=jax.ShapeDtypeStruct((E, MUL), jnp.float32),
        mesh=_SC_MESH,
        compiler_params=_SC_PARAMS,
        scratch_types=[
            pltpu.VMEM((EPW,), jnp.int32),
            pltpu.VMEM((EPW, MUL), jnp.float32),
            pltpu.SemaphoreType.DMA,
        ],
    )
    def k(table_hbm, idx_hbm, out_hbm, idx_v, rows_v, sem):
        wid = lax.axis_index("s") * 2 + lax.axis_index("c")
        base = wid * EPW
        pltpu.sync_copy(idx_hbm.at[pl.ds(base, EPW)], idx_v)
        pltpu.async_copy(table_hbm.at[idx_v], rows_v, sem).wait()
        pltpu.sync_copy(rows_v, out_hbm.at[pl.ds(base, EPW)])

    return k(table, idx)


def _sc_scatter(msg, idx):
    @functools.partial(
        pl.kernel,
        out_type=jax.ShapeDtypeStruct((2, N, MUL), jnp.float32),
        mesh=_SC_MESH,
        compiler_params=_SC_PARAMS,
        scratch_types=[
            pltpu.VMEM((EPW,), jnp.int32),
            pltpu.VMEM((EPW, MUL), jnp.float32),
            pltpu.VMEM((NPT, MUL), jnp.float32),
            pltpu.VMEM_SHARED((N, MUL), jnp.float32),
            pltpu.SemaphoreType.DMA,
        ],
    )
    def k(msg_hbm, idx_hbm, out_hbm, idx_v, msg_v, node_v, acc_sh, sem):
        c = lax.axis_index("c")
        s = lax.axis_index("s")
        wid = s * 2 + c

        def zrow(i, _):
            node_v[i, :] = jnp.zeros((MUL,), jnp.float32)
            return 0
        lax.fori_loop(0, NPT, zrow, 0)
        pltpu.sync_copy(node_v, acc_sh.at[pl.ds(s * NPT, NPT)])
        plsc.subcore_barrier()

        base = wid * EPW
        pltpu.sync_copy(idx_hbm.at[pl.ds(base, EPW)], idx_v)
        pltpu.sync_copy(msg_hbm.at[pl.ds(base, EPW)], msg_v)
        pltpu.sync_copy(msg_v, acc_sh.at[idx_v], add=True)
        plsc.subcore_barrier()

        pltpu.sync_copy(acc_sh.at[pl.ds(s * NPT, NPT)], node_v)
        pltpu.sync_copy(node_v, out_hbm.at[c, pl.ds(s * NPT, NPT)])

    return k(msg, idx)


def _edge_body(ps_ref, pd_ref, hsrc_ref, w1_ref, b1_ref, w2_ref, b2_ref,
               w3_ref, b3_ref, msg_ref):
    rel = pd_ref[:, 0:3] - ps_ref[:, 0:3]
    d2 = jnp.sum(rel * rel, axis=1, keepdims=True)
    dist = jnp.maximum(jnp.sqrt(d2), 1e-6)
    centers = (jax.lax.broadcasted_iota(jnp.int32, (1, RB), 1)
               .astype(jnp.float32) * np.float32(6.0 / (RB - 1)))
    rbf = jnp.exp(-((dist - centers) ** 2) * 2.0) * _RBF_NORM
    hh = _silu(jnp.dot(rbf, w1_ref[...], preferred_element_type=jnp.float32)
               + b1_ref[...])
    hh = _silu(jnp.dot(hh, w2_ref[...], preferred_element_type=jnp.float32)
               + b2_ref[...])
    tw = (jnp.dot(hh, w3_ref[...], preferred_element_type=jnp.float32)
          + b3_ref[...])
    hsrc = hsrc_ref[...]
    acc = tw[:, 0:MUL] * hsrc[:, 0:1]
    for u in range(1, MUL):
        acc = acc + tw[:, u * MUL:(u + 1) * MUL] * hsrc[:, u:u + 1]
    msg_ref[...] = acc * np.float32(1.0 / np.sqrt(MUL))


def _edge_msg(possrc, posdst, hsrc, w1, b1, w2, b2, w3, b3):
    grid = (E // TE,)
    return pl.pallas_call(
        _edge_body,
        grid=grid,
        in_specs=[
            pl.BlockSpec((TE, MUL), lambda i: (i, 0)),
            pl.BlockSpec((TE, MUL), lambda i: (i, 0)),
            pl.BlockSpec((TE, MUL), lambda i: (i, 0)),
            pl.BlockSpec((RB, RH), lambda i: (0, 0)),
            pl.BlockSpec((1, RH), lambda i: (0, 0)),
            pl.BlockSpec((RH, RH), lambda i: (0, 0)),
            pl.BlockSpec((1, RH), lambda i: (0, 0)),
            pl.BlockSpec((RH, MUL * MUL), lambda i: (0, 0)),
            pl.BlockSpec((1, MUL * MUL), lambda i: (0, 0)),
        ],
        out_specs=pl.BlockSpec((TE, MUL), lambda i: (i, 0)),
        out_shape=jax.ShapeDtypeStruct((E, MUL), jnp.float32),
    )(possrc, posdst, hsrc, w1, b1.reshape(1, RH), w2, b2.reshape(1, RH),
      w3, b3.reshape(1, MUL * MUL))


def _matmul_body(x_ref, w_ref, o_ref):
    o_ref[...] = jnp.dot(x_ref[...], w_ref[...],
                         preferred_element_type=jnp.float32)


def _matmul(x, w):
    m, _ = x.shape
    _, n = w.shape
    return pl.pallas_call(
        _matmul_body,
        out_shape=jax.ShapeDtypeStruct((m, n), jnp.float32),
    )(x, w)


def _update_body(h_ref, w_ref, p_ref, o_ref):
    o_ref[...] = (jnp.dot(h_ref[...], w_ref[...],
                          preferred_element_type=jnp.float32)
                  + p_ref[0] + p_ref[1])


def _self_plus_agg(h, w, parts):
    return pl.pallas_call(
        _update_body,
        out_shape=jax.ShapeDtypeStruct(h.shape, jnp.float32),
    )(h, w, parts)


def _tail_body(h_ref, batch_ref, p1w_ref, p1b_ref, p2w_ref, p2b_ref,
               p3w_ref, p3b_ref, ow_ref, ob_ref, lng_ref, lnb_ref, out_ref):
    h = h_ref[...]
    a = _silu(jnp.dot(h, p1w_ref[...], preferred_element_type=jnp.float32)
              + p1b_ref[...])
    a = _silu(jnp.dot(a, p2w_ref[...], preferred_element_type=jnp.float32)
              + p2b_ref[...])
    logits = (jnp.dot(a, p3w_ref[...], preferred_element_type=jnp.float32)
              + p3b_ref[...])
    batch = batch_ref[...]
    seg_ids = jax.lax.broadcasted_iota(jnp.int32, (N, NB), 1)
    onehot = batch == seg_ids
    onehot_f = onehot.astype(jnp.float32)
    masked = jnp.where(onehot, logits, jnp.float32(-3e38))
    seg_max = jnp.max(masked, axis=0, keepdims=True)
    maxn = jnp.sum(onehot_f * seg_max, axis=1, keepdims=True)
    zexp = jnp.exp(logits - maxn)
    denom = jnp.sum(onehot_f * zexp, axis=0, keepdims=True)
    wgt = onehot_f * zexp
    g = jax.lax.dot_general(wgt, h, (((0,), (0,)), ((), ())),
                            preferred_element_type=jnp.float32)
    g = g / denom.reshape(NB, 1)
    o = (jnp.dot(g, ow_ref[...], preferred_element_type=jnp.float32)
         + ob_ref[...])
    mu = jnp.mean(o, axis=1, keepdims=True)
    var = jnp.mean(o * o, axis=1, keepdims=True) - mu * mu
    out_ref[...] = ((o - mu) * jax.lax.rsqrt(var + 1e-5)
                    * lng_ref[...] + lnb_ref[...])


def _tail(h, batch, params):
    return pl.pallas_call(
        _tail_body,
        out_shape=jax.ShapeDtypeStruct((NB, OUT_DIM), jnp.float32),
    )(h, batch.reshape(N, 1),
      params['p1_w'], params['p1_b'].reshape(1, POOL_H),
      params['p2_w'], params['p2_b'].reshape(1, POOL_H),
      params['p3_w'], params['p3_b'].reshape(1, 1),
      params['o_w'], params['o_b'].reshape(1, OUT_DIM),
      params['ln_g'].reshape(1, OUT_DIM), params['ln_b'].reshape(1, OUT_DIM))


def kernel(x, pos, edge_index, batch, params):
    src = edge_index[0]
    dst = edge_index[1]
    h = _matmul(x, params['W_in'] / np.sqrt(D_IN))
    pos_pad = jnp.pad(pos, ((0, 0), (0, MUL - 3)))
    possrc = _sc_gather(pos_pad, src)
    posdst = _sc_gather(pos_pad, dst)
    for lp in params['layers']:
        hsrc = _sc_gather(h, src)
        msg = _edge_msg(possrc, posdst, hsrc,
                        lp['w1'], lp['b1'], lp['w2'], lp['b2'],
                        lp['w3'], lp['b3'])
        parts = _sc_scatter(msg, dst)
        h = _self_plus_agg(h, lp['W_si'] / np.sqrt(MUL), parts)
    return _tail(h, batch, params)


# speedup vs baseline: 4.9856x; 2.5658x over previous
import functools

import numpy as np
import jax
import jax.numpy as jnp
from jax import lax
from jax.experimental import pallas as pl
from jax.experimental.pallas import tpu as pltpu
from jax.experimental.pallas import tpu_sc as plsc

N = 10000
E = 160000
D_IN = 128
MUL = 16
RB = 8
RH = 32
POOL_H = 128
OUT_DIM = 512
NB = 16

TE = 3200
NW = 32
EPW = E // NW
NPT = N // 16

def _sc_mesh():
    return plsc.VectorSubcoreMesh(core_axis_name="c", subcore_axis_name="s")


_SC_PARAMS = pltpu.CompilerParams(use_tc_tiling_on_sc=False)
_RBF_NORM = np.float32(1.0 / (np.sqrt(2.0 * 3.14159) * 0.5))


def _silu(v):
    return v * jax.nn.sigmoid(v)


def _sc_gather(table, idx):
    @functools.partial(
        pl.kernel,
        out_type=jax.ShapeDtypeStruct((E, MUL), jnp.float32),
        mesh=_sc_mesh(),
        compiler_params=_SC_PARAMS,
        scratch_types=[
            pltpu.VMEM((EPW,), jnp.int32),
            pltpu.VMEM((EPW, MUL), jnp.float32),
            pltpu.SemaphoreType.DMA,
        ],
    )
    def k(table_hbm, idx_hbm, out_hbm, idx_v, rows_v, sem):
        wid = lax.axis_index("s") * 2 + lax.axis_index("c")
        base = wid * EPW
        pltpu.sync_copy(idx_hbm.at[pl.ds(base, EPW)], idx_v)
        pltpu.async_copy(table_hbm.at[idx_v], rows_v, sem).wait()
        pltpu.sync_copy(rows_v, out_hbm.at[pl.ds(base, EPW)])

    return k(table, idx)


def _sc_scatter(msg, idx):
    @functools.partial(
        pl.kernel,
        out_type=jax.ShapeDtypeStruct((2, N, MUL), jnp.float32),
        mesh=_sc_mesh(),
        compiler_params=_SC_PARAMS,
        scratch_types=[
            pltpu.VMEM((EPW,), jnp.int32),
            pltpu.VMEM((EPW, MUL), jnp.float32),
            pltpu.VMEM((NPT, MUL), jnp.float32),
            pltpu.VMEM_SHARED((N, MUL), jnp.float32),
            pltpu.SemaphoreType.DMA,
        ],
    )
    def k(msg_hbm, idx_hbm, out_hbm, idx_v, msg_v, node_v, acc_sh, sem):
        c = lax.axis_index("c")
        s = lax.axis_index("s")
        wid = s * 2 + c

        def zrow(i, _):
            node_v[i, :] = jnp.zeros((MUL,), jnp.float32)
            return 0
        lax.fori_loop(0, NPT, zrow, 0)
        pltpu.sync_copy(node_v, acc_sh.at[pl.ds(s * NPT, NPT)])
        plsc.subcore_barrier()

        base = wid * EPW
        pltpu.sync_copy(idx_hbm.at[pl.ds(base, EPW)], idx_v)
        pltpu.sync_copy(msg_hbm.at[pl.ds(base, EPW)], msg_v)
        pltpu.sync_copy(msg_v, acc_sh.at[idx_v], add=True)
        plsc.subcore_barrier()

        pltpu.sync_copy(acc_sh.at[pl.ds(s * NPT, NPT)], node_v)
        pltpu.sync_copy(node_v, out_hbm.at[c, pl.ds(s * NPT, NPT)])

    return k(msg, idx)


_Q = np.zeros((MUL, MUL * MUL), np.float32)
for _u in range(MUL):
    _Q[_u, _u * MUL:(_u + 1) * MUL] = 1.0
_R = np.zeros((MUL * MUL, MUL), np.float32)
for _u in range(MUL):
    for _v in range(MUL):
        _R[_u * MUL + _v, _v] = 1.0


def _edge_body(ps_ref, pd_ref, hsrc_ref, w1_ref, b1_ref, w2_ref, b2_ref,
               w3_ref, b3_ref, q_ref, r_ref, msg_ref):
    ps = ps_ref[...]
    pd = pd_ref[...]
    hsrc = hsrc_ref[...]
    rel = pd[:, 0:3] - ps[:, 0:3]
    d2 = jnp.sum(rel * rel, axis=1, keepdims=True)
    dist = jnp.maximum(jnp.sqrt(d2), 1e-6)
    centers = (jax.lax.broadcasted_iota(jnp.int32, (1, RB), 1)
               .astype(jnp.float32) * np.float32(6.0 / (RB - 1)))
    rbf = jnp.exp(-((dist - centers) ** 2) * 2.0) * _RBF_NORM
    hh = _silu(jnp.dot(rbf, w1_ref[...], preferred_element_type=jnp.float32)
               + b1_ref[...])
    hh = _silu(jnp.dot(hh, w2_ref[...], preferred_element_type=jnp.float32)
               + b2_ref[...])
    tw = (jnp.dot(hh, w3_ref[...], preferred_element_type=jnp.float32)
          + b3_ref[...])
    hbig = jnp.dot(hsrc, q_ref[...], preferred_element_type=jnp.float32)
    msg = jnp.dot(tw * hbig, r_ref[...],
                  preferred_element_type=jnp.float32)
    msg_ref[...] = msg * np.float32(1.0 / np.sqrt(MUL))


def _edge_msg(possrc, posdst, hsrc, w1, b1, w2, b2, w3, b3):
    grid = (E // TE,)
    return pl.pallas_call(
        _edge_body,
        grid=grid,
        in_specs=[
            pl.BlockSpec((TE, MUL), lambda i: (i, 0)),
            pl.BlockSpec((TE, MUL), lambda i: (i, 0)),
            pl.BlockSpec((TE, MUL), lambda i: (i, 0)),
            pl.BlockSpec((RB, RH), lambda i: (0, 0)),
            pl.BlockSpec((1, RH), lambda i: (0, 0)),
            pl.BlockSpec((RH, RH), lambda i: (0, 0)),
            pl.BlockSpec((1, RH), lambda i: (0, 0)),
            pl.BlockSpec((RH, MUL * MUL), lambda i: (0, 0)),
            pl.BlockSpec((1, MUL * MUL), lambda i: (0, 0)),
            pl.BlockSpec((MUL, MUL * MUL), lambda i: (0, 0)),
            pl.BlockSpec((MUL * MUL, MUL), lambda i: (0, 0)),
        ],
        out_specs=pl.BlockSpec((TE, MUL), lambda i: (i, 0)),
        out_shape=jax.ShapeDtypeStruct((E, MUL), jnp.float32),
    )(possrc, posdst, hsrc,
      w1, b1.reshape(1, RH), w2, b2.reshape(1, RH),
      w3, b3.reshape(1, MUL * MUL), jnp.asarray(_Q), jnp.asarray(_R))


def _matmul_body(x_ref, w_ref, o_ref):
    o_ref[...] = jnp.dot(x_ref[...], w_ref[...],
                         preferred_element_type=jnp.float32)


def _matmul(x, w):
    m, _ = x.shape
    _, n = w.shape
    return pl.pallas_call(
        _matmul_body,
        out_shape=jax.ShapeDtypeStruct((m, n), jnp.float32),
    )(x, w)


def _update_body(h_ref, w_ref, p_ref, o_ref):
    o_ref[...] = (jnp.dot(h_ref[...], w_ref[...],
                          preferred_element_type=jnp.float32)
                  + p_ref[0] + p_ref[1])


def _self_plus_agg(h, w, parts):
    return pl.pallas_call(
        _update_body,
        out_shape=jax.ShapeDtypeStruct(h.shape, jnp.float32),
    )(h, w, parts)


def _tail_body(h_ref, batch_ref, p1w_ref, p1b_ref, p2w_ref, p2b_ref,
               p3w_ref, p3b_ref, ow_ref, ob_ref, lng_ref, lnb_ref, out_ref):
    h = h_ref[...]
    a = _silu(jnp.dot(h, p1w_ref[...], preferred_element_type=jnp.float32)
              + p1b_ref[...])
    a = _silu(jnp.dot(a, p2w_ref[...], preferred_element_type=jnp.float32)
              + p2b_ref[...])
    logits = (jnp.dot(a, p3w_ref[...], preferred_element_type=jnp.float32)
              + p3b_ref[...])
    batch = batch_ref[...]
    seg_ids = jax.lax.broadcasted_iota(jnp.int32, (N, NB), 1)
    onehot = batch == seg_ids
    onehot_f = onehot.astype(jnp.float32)
    masked = jnp.where(onehot, logits, jnp.float32(-3e38))
    seg_max = jnp.max(masked, axis=0, keepdims=True)
    maxn = jnp.sum(onehot_f * seg_max, axis=1, keepdims=True)
    zexp = jnp.exp(logits - maxn)
    denom = jnp.sum(onehot_f * zexp, axis=0, keepdims=True)
    wgt = onehot_f * zexp
    g = jax.lax.dot_general(wgt, h, (((0,), (0,)), ((), ())),
                            preferred_element_type=jnp.float32)
    g = g / denom.reshape(NB, 1)
    o = (jnp.dot(g, ow_ref[...], preferred_element_type=jnp.float32)
         + ob_ref[...])
    mu = jnp.mean(o, axis=1, keepdims=True)
    var = jnp.mean(o * o, axis=1, keepdims=True) - mu * mu
    out_ref[...] = ((o - mu) * jax.lax.rsqrt(var + 1e-5)
                    * lng_ref[...] + lnb_ref[...])


def _tail(h, batch, params):
    return pl.pallas_call(
        _tail_body,
        out_shape=jax.ShapeDtypeStruct((NB, OUT_DIM), jnp.float32),
    )(h, batch.reshape(N, 1),
      params['p1_w'], params['p1_b'].reshape(1, POOL_H),
      params['p2_w'], params['p2_b'].reshape(1, POOL_H),
      params['p3_w'], params['p3_b'].reshape(1, 1),
      params['o_w'], params['o_b'].reshape(1, OUT_DIM),
      params['ln_g'].reshape(1, OUT_DIM), params['ln_b'].reshape(1, OUT_DIM))


def kernel(x, pos, edge_index, batch, params):
    src = edge_index[0]
    dst = edge_index[1]
    h = _matmul(x, params['W_in'] / np.sqrt(D_IN))
    pos_pad = jnp.pad(pos, ((0, 0), (0, MUL - 3)))
    possrc = _sc_gather(pos_pad, src)
    posdst = _sc_gather(pos_pad, dst)
    for lp in params['layers']:
        hsrc = _sc_gather(h, src)
        msg = _edge_msg(possrc, posdst, hsrc,
                        lp['w1'], lp['b1'], lp['w2'], lp['b2'],
                        lp['w3'], lp['b3'])
        parts = _sc_scatter(msg, dst)
        h = _self_plus_agg(h, lp['W_si'] / np.sqrt(MUL), parts)
    return _tail(h, batch, params)


# speedup vs baseline: 5.3252x; 1.0681x over previous
import functools

import numpy as np
import jax
import jax.numpy as jnp
from jax import lax
from jax.experimental import pallas as pl
from jax.experimental.pallas import tpu as pltpu
from jax.experimental.pallas import tpu_sc as plsc

N = 10000
E = 160000
D_IN = 128
MUL = 16
RB = 8
RH = 32
POOL_H = 128
OUT_DIM = 512
NB = 16

TE = 3200
NW = 32
EPW = E // NW
NPT = N // 16

def _sc_mesh():
    return plsc.VectorSubcoreMesh(core_axis_name="c", subcore_axis_name="s")


_SC_PARAMS = pltpu.CompilerParams(use_tc_tiling_on_sc=False)
_RBF_NORM = np.float32(1.0 / (np.sqrt(2.0 * 3.14159) * 0.5))


def _silu(v):
    return v * jax.nn.sigmoid(v)


def _sc_gather(table, idx):
    @functools.partial(
        pl.kernel,
        out_type=jax.ShapeDtypeStruct((E, MUL), jnp.float32),
        mesh=_sc_mesh(),
        compiler_params=_SC_PARAMS,
        scratch_types=[
            pltpu.VMEM((EPW,), jnp.int32),
            pltpu.VMEM((EPW, MUL), jnp.float32),
            pltpu.SemaphoreType.DMA,
        ],
    )
    def k(table_hbm, idx_hbm, out_hbm, idx_v, rows_v, sem):
        wid = lax.axis_index("s") * 2 + lax.axis_index("c")
        base = wid * EPW
        pltpu.sync_copy(idx_hbm.at[pl.ds(base, EPW)], idx_v)
        pltpu.async_copy(table_hbm.at[idx_v], rows_v, sem).wait()
        pltpu.sync_copy(rows_v, out_hbm.at[pl.ds(base, EPW)])

    return k(table, idx)


def _sc_scatter(msg, idx):
    @functools.partial(
        pl.kernel,
        out_type=jax.ShapeDtypeStruct((2, N, MUL), jnp.float32),
        mesh=_sc_mesh(),
        compiler_params=_SC_PARAMS,
        scratch_types=[
            pltpu.VMEM((EPW,), jnp.int32),
            pltpu.VMEM((EPW, MUL), jnp.float32),
            pltpu.VMEM((NPT, MUL), jnp.float32),
            pltpu.VMEM_SHARED((N, MUL), jnp.float32),
            pltpu.SemaphoreType.DMA,
        ],
    )
    def k(msg_hbm, idx_hbm, out_hbm, idx_v, msg_v, node_v, acc_sh, sem):
        c = lax.axis_index("c")
        s = lax.axis_index("s")
        wid = s * 2 + c

        def zrow(i, _):
            node_v[i, :] = jnp.zeros((MUL,), jnp.float32)
            return 0
        lax.fori_loop(0, NPT, zrow, 0)
        pltpu.sync_copy(node_v, acc_sh.at[pl.ds(s * NPT, NPT)])
        plsc.subcore_barrier()

        base = wid * EPW
        pltpu.sync_copy(idx_hbm.at[pl.ds(base, EPW)], idx_v)
        pltpu.sync_copy(msg_hbm.at[pl.ds(base, EPW)], msg_v)
        pltpu.sync_copy(msg_v, acc_sh.at[idx_v], add=True)
        plsc.subcore_barrier()

        pltpu.sync_copy(acc_sh.at[pl.ds(s * NPT, NPT)], node_v)
        pltpu.sync_copy(node_v, out_hbm.at[c, pl.ds(s * NPT, NPT)])

    return k(msg, idx)


_Q = np.zeros((MUL, MUL * MUL), np.float32)
for _u in range(MUL):
    _Q[_u, _u * MUL:(_u + 1) * MUL] = 1.0
_R = np.zeros((MUL * MUL, MUL), np.float32)
for _u in range(MUL):
    for _v in range(MUL):
        _R[_u * MUL + _v, _v] = 1.0


def _edge_body(ps_ref, pd_ref, hsrc_ref, w1_ref, b1_ref, w2_ref, b2_ref,
               w3_ref, b3_ref, q_ref, r_ref, msg_ref):
    ps8 = ps_ref[...]
    pd8 = pd_ref[...]
    hs8 = hsrc_ref[...]
    rel = jnp.concatenate(
        [pd8[:, 16 * j:16 * j + 3] - ps8[:, 16 * j:16 * j + 3]
         for j in range(8)], axis=0)
    hsrc = jnp.concatenate(
        [hs8[:, 16 * j:16 * (j + 1)] for j in range(8)], axis=0)
    d2 = jnp.sum(rel * rel, axis=1, keepdims=True)
    dist = jnp.maximum(jnp.sqrt(d2), 1e-6)
    centers = (jax.lax.broadcasted_iota(jnp.int32, (1, RB), 1)
               .astype(jnp.float32) * np.float32(6.0 / (RB - 1)))
    rbf = jnp.exp(-((dist - centers) ** 2) * 2.0) * _RBF_NORM
    hh = _silu(jnp.dot(rbf, w1_ref[...], preferred_element_type=jnp.float32)
               + b1_ref[...])
    hh = _silu(jnp.dot(hh, w2_ref[...], preferred_element_type=jnp.float32)
               + b2_ref[...])
    tw = (jnp.dot(hh, w3_ref[...], preferred_element_type=jnp.float32)
          + b3_ref[...])
    hbig = jnp.dot(hsrc, q_ref[...], preferred_element_type=jnp.float32)
    msg = jnp.dot(tw * hbig, r_ref[...],
                  preferred_element_type=jnp.float32)
    msg = msg * np.float32(1.0 / np.sqrt(MUL))
    ts = TE // 8
    msg_ref[...] = jnp.concatenate(
        [msg[ts * j:ts * (j + 1), :] for j in range(8)], axis=1)


def _edge_msg(possrc, posdst, hsrc, w1, b1, w2, b2, w3, b3):
    grid = (E // TE,)
    pk = lambda a: a.reshape(E // 8, 128)
    return pl.pallas_call(
        _edge_body,
        grid=grid,
        in_specs=[
            pl.BlockSpec((TE // 8, 128), lambda i: (i, 0)),
            pl.BlockSpec((TE // 8, 128), lambda i: (i, 0)),
            pl.BlockSpec((TE // 8, 128), lambda i: (i, 0)),
            pl.BlockSpec((RB, RH), lambda i: (0, 0)),
            pl.BlockSpec((1, RH), lambda i: (0, 0)),
            pl.BlockSpec((RH, RH), lambda i: (0, 0)),
            pl.BlockSpec((1, RH), lambda i: (0, 0)),
            pl.BlockSpec((RH, MUL * MUL), lambda i: (0, 0)),
            pl.BlockSpec((1, MUL * MUL), lambda i: (0, 0)),
            pl.BlockSpec((MUL, MUL * MUL), lambda i: (0, 0)),
            pl.BlockSpec((MUL * MUL, MUL), lambda i: (0, 0)),
        ],
        out_specs=pl.BlockSpec((TE // 8, 128), lambda i: (i, 0)),
        out_shape=jax.ShapeDtypeStruct((E // 8, 128), jnp.float32),
    )(pk(possrc), pk(posdst), pk(hsrc),
      w1, b1.reshape(1, RH), w2, b2.reshape(1, RH),
      w3, b3.reshape(1, MUL * MUL), jnp.asarray(_Q), jnp.asarray(_R))


def _matmul_body(x_ref, w_ref, o_ref):
    o_ref[...] = jnp.dot(x_ref[...], w_ref[...],
                         preferred_element_type=jnp.float32)


def _matmul(x, w):
    m, _ = x.shape
    _, n = w.shape
    return pl.pallas_call(
        _matmul_body,
        out_shape=jax.ShapeDtypeStruct((m, n), jnp.float32),
    )(x, w)


def _update_body(h_ref, w_ref, p_ref, o_ref):
    o_ref[...] = (jnp.dot(h_ref[...], w_ref[...],
                          preferred_element_type=jnp.float32)
                  + p_ref[0] + p_ref[1])


def _self_plus_agg(h, w, parts):
    return pl.pallas_call(
        _update_body,
        out_shape=jax.ShapeDtypeStruct(h.shape, jnp.float32),
    )(h, w, parts)


def _tail_body(h_ref, batch_ref, p1w_ref, p1b_ref, p2w_ref, p2b_ref,
               p3w_ref, p3b_ref, ow_ref, ob_ref, lng_ref, lnb_ref, out_ref):
    h = h_ref[...]
    a = _silu(jnp.dot(h, p1w_ref[...], preferred_element_type=jnp.float32)
              + p1b_ref[...])
    a = _silu(jnp.dot(a, p2w_ref[...], preferred_element_type=jnp.float32)
              + p2b_ref[...])
    logits = (jnp.dot(a, p3w_ref[...], preferred_element_type=jnp.float32)
              + p3b_ref[...])
    batch = batch_ref[...]
    seg_ids = jax.lax.broadcasted_iota(jnp.int32, (N, NB), 1)
    onehot = batch == seg_ids
    onehot_f = onehot.astype(jnp.float32)
    masked = jnp.where(onehot, logits, jnp.float32(-3e38))
    seg_max = jnp.max(masked, axis=0, keepdims=True)
    maxn = jnp.sum(onehot_f * seg_max, axis=1, keepdims=True)
    zexp = jnp.exp(logits - maxn)
    denom = jnp.sum(onehot_f * zexp, axis=0, keepdims=True)
    wgt = onehot_f * zexp
    g = jax.lax.dot_general(wgt, h, (((0,), (0,)), ((), ())),
                            preferred_element_type=jnp.float32)
    g = g / denom.reshape(NB, 1)
    o = (jnp.dot(g, ow_ref[...], preferred_element_type=jnp.float32)
         + ob_ref[...])
    mu = jnp.mean(o, axis=1, keepdims=True)
    var = jnp.mean(o * o, axis=1, keepdims=True) - mu * mu
    out_ref[...] = ((o - mu) * jax.lax.rsqrt(var + 1e-5)
                    * lng_ref[...] + lnb_ref[...])


def _tail(h, batch, params):
    return pl.pallas_call(
        _tail_body,
        out_shape=jax.ShapeDtypeStruct((NB, OUT_DIM), jnp.float32),
    )(h, batch.reshape(N, 1),
      params['p1_w'], params['p1_b'].reshape(1, POOL_H),
      params['p2_w'], params['p2_b'].reshape(1, POOL_H),
      params['p3_w'], params['p3_b'].reshape(1, 1),
      params['o_w'], params['o_b'].reshape(1, OUT_DIM),
      params['ln_g'].reshape(1, OUT_DIM), params['ln_b'].reshape(1, OUT_DIM))


def kernel(x, pos, edge_index, batch, params):
    src = edge_index[0]
    dst = edge_index[1]
    h = _matmul(x, params['W_in'] / np.sqrt(D_IN))
    pos_pad = jnp.pad(pos, ((0, 0), (0, MUL - 3)))
    possrc = _sc_gather(pos_pad, src)
    posdst = _sc_gather(pos_pad, dst)
    for lp in params['layers']:
        hsrc = _sc_gather(h, src)
        msg = _edge_msg(possrc, posdst, hsrc,
                        lp['w1'], lp['b1'], lp['w2'], lp['b2'],
                        lp['w3'], lp['b3'])
        parts = _sc_scatter(msg.reshape(E, MUL), dst)
        h = _self_plus_agg(h, lp['W_si'] / np.sqrt(MUL), parts)
    return _tail(h, batch, params)


# speedup vs baseline: 8.8772x; 1.6670x over previous
import functools

import numpy as np
import jax
import jax.numpy as jnp
from jax import lax
from jax.experimental import pallas as pl
from jax.experimental.pallas import tpu as pltpu
from jax.experimental.pallas import tpu_sc as plsc

N = 10000
E = 160000
D_IN = 128
MUL = 16
RB = 8
RH = 32
POOL_H = 128
OUT_DIM = 512
NB = 16

TE = 3200
NW = 32
EPW = E // NW
NPT = N // 16

def _sc_mesh():
    return plsc.VectorSubcoreMesh(core_axis_name="c", subcore_axis_name="s")


_SC_PARAMS = pltpu.CompilerParams(use_tc_tiling_on_sc=False)
_RBF_NORM = np.float32(1.0 / (np.sqrt(2.0 * 3.14159) * 0.5))


def _silu(v):
    return v * jax.nn.sigmoid(v)


def _sc_gather(table, idx):
    @functools.partial(
        pl.kernel,
        out_type=jax.ShapeDtypeStruct((E, MUL), jnp.float32),
        mesh=_sc_mesh(),
        compiler_params=_SC_PARAMS,
        scratch_types=[
            pltpu.VMEM((EPW,), jnp.int32),
            pltpu.VMEM((EPW, MUL), jnp.float32),
            pltpu.SemaphoreType.DMA,
        ],
    )
    def k(table_hbm, idx_hbm, out_hbm, idx_v, rows_v, sem):
        wid = lax.axis_index("s") * 2 + lax.axis_index("c")
        base = wid * EPW
        pltpu.sync_copy(idx_hbm.at[pl.ds(base, EPW)], idx_v)
        pltpu.async_copy(table_hbm.at[idx_v], rows_v, sem).wait()
        pltpu.sync_copy(rows_v, out_hbm.at[pl.ds(base, EPW)])

    return k(table, idx)


def _sc_scatter(msg, idx):
    @functools.partial(
        pl.kernel,
        out_type=jax.ShapeDtypeStruct((2, N, MUL), jnp.float32),
        mesh=_sc_mesh(),
        compiler_params=_SC_PARAMS,
        scratch_types=[
            pltpu.VMEM((EPW,), jnp.int32),
            pltpu.VMEM((EPW, MUL), jnp.float32),
            pltpu.VMEM((NPT, MUL), jnp.float32),
            pltpu.VMEM_SHARED((N, MUL), jnp.float32),
            pltpu.SemaphoreType.DMA,
        ],
    )
    def k(msg_hbm, idx_hbm, out_hbm, idx_v, msg_v, node_v, acc_sh, sem):
        c = lax.axis_index("c")
        s = lax.axis_index("s")
        wid = s * 2 + c

        def zrow(i, _):
            node_v[i, :] = jnp.zeros((MUL,), jnp.float32)
            return 0
        lax.fori_loop(0, NPT, zrow, 0)
        pltpu.sync_copy(node_v, acc_sh.at[pl.ds(s * NPT, NPT)])
        plsc.subcore_barrier()

        base = wid * EPW
        pltpu.sync_copy(idx_hbm.at[pl.ds(base, EPW)], idx_v)
        pltpu.sync_copy(msg_hbm.at[pl.ds(base, EPW)], msg_v)
        pltpu.sync_copy(msg_v, acc_sh.at[idx_v], add=True)
        plsc.subcore_barrier()

        pltpu.sync_copy(acc_sh.at[pl.ds(s * NPT, NPT)], node_v)
        pltpu.sync_copy(node_v, out_hbm.at[c, pl.ds(s * NPT, NPT)])

    return k(msg, idx)


_Q = np.zeros((MUL, MUL * MUL), np.float32)
for _u in range(MUL):
    _Q[_u, _u * MUL:(_u + 1) * MUL] = 1.0
_R = np.zeros((MUL * MUL, MUL), np.float32)
for _u in range(MUL):
    for _v in range(MUL):
        _R[_u * MUL + _v, _v] = 1.0


_S = np.zeros((128, 128), np.float32)
for _j in range(8):
    for _k in range(3):
        for _i in range(RB):
            _S[16 * _j + _k, 16 * _j + _i] = 1.0


def _edge_body(ps_ref, pd_ref, hsrc_ref, w1bd_ref, b1bd_ref, w2bd_ref,
               b2bd_ref, w3_ref, b3_ref, q_ref, r_ref, s_ref, msg_ref):
    ps8 = ps_ref[...]
    pd8 = pd_ref[...]
    hs8 = hsrc_ref[...]
    diff = pd8 - ps8
    d2 = jnp.dot(diff * diff, s_ref[...],
                 preferred_element_type=jnp.float32)
    dist = jnp.maximum(jnp.sqrt(d2), 1e-6)
    li = jax.lax.broadcasted_iota(jnp.int32, (1, 128), 1) % 16
    centers = jnp.where(li < RB, li.astype(jnp.float32)
                        * np.float32(6.0 / (RB - 1)), jnp.float32(1e5))
    rbf = jnp.exp(-((dist - centers) ** 2) * 2.0) * _RBF_NORM
    h1 = _silu(jnp.dot(rbf, w1bd_ref[...],
                       preferred_element_type=jnp.float32) + b1bd_ref[...])
    h2 = _silu(jnp.dot(h1, w2bd_ref[...],
                       preferred_element_type=jnp.float32) + b2bd_ref[...])
    w3 = w3_ref[...]
    b3 = b3_ref[...]
    q = q_ref[...]
    r = r_ref[...]
    msgs = []
    for j in range(8):
        twj = (jnp.dot(h2[:, 32 * j:32 * (j + 1)], w3,
                       preferred_element_type=jnp.float32) + b3)
        hbj = jnp.dot(hs8[:, 16 * j:16 * (j + 1)], q,
                      preferred_element_type=jnp.float32)
        msgs.append(jnp.dot(twj * hbj, r,
                            preferred_element_type=jnp.float32))
    msg_ref[...] = (jnp.concatenate(msgs, axis=1)
                    * np.float32(1.0 / np.sqrt(MUL)))


def _edge_msg(possrc, posdst, hsrc, w1, b1, w2, b2, w3, b3):
    grid = (E // TE,)
    pk = lambda a: a.reshape(E // 8, 128)
    eye8 = jnp.eye(8, dtype=jnp.float32)
    w1bd = jnp.kron(eye8, jnp.pad(w1, ((0, MUL - RB), (0, 0))))
    w2bd = jnp.kron(eye8, w2)
    b1bd = jnp.tile(b1, 8).reshape(1, 8 * RH)
    b2bd = jnp.tile(b2, 8).reshape(1, 8 * RH)
    return pl.pallas_call(
        _edge_body,
        grid=grid,
        in_specs=[
            pl.BlockSpec((TE // 8, 128), lambda i: (i, 0)),
            pl.BlockSpec((TE // 8, 128), lambda i: (i, 0)),
            pl.BlockSpec((TE // 8, 128), lambda i: (i, 0)),
            pl.BlockSpec((128, 8 * RH), lambda i: (0, 0)),
            pl.BlockSpec((1, 8 * RH), lambda i: (0, 0)),
            pl.BlockSpec((8 * RH, 8 * RH), lambda i: (0, 0)),
            pl.BlockSpec((1, 8 * RH), lambda i: (0, 0)),
            pl.BlockSpec((RH, MUL * MUL), lambda i: (0, 0)),
            pl.BlockSpec((1, MUL * MUL), lambda i: (0, 0)),
            pl.BlockSpec((MUL, MUL * MUL), lambda i: (0, 0)),
            pl.BlockSpec((MUL * MUL, MUL), lambda i: (0, 0)),
            pl.BlockSpec((128, 128), lambda i: (0, 0)),
        ],
        out_specs=pl.BlockSpec((TE // 8, 128), lambda i: (i, 0)),
        out_shape=jax.ShapeDtypeStruct((E // 8, 128), jnp.float32),
    )(pk(possrc), pk(posdst), pk(hsrc),
      w1bd, b1bd, w2bd, b2bd,
      w3, b3.reshape(1, MUL * MUL), jnp.asarray(_Q), jnp.asarray(_R),
      jnp.asarray(_S))


def _matmul_body(x_ref, w_ref, o_ref):
    o_ref[...] = jnp.dot(x_ref[...], w_ref[...],
                         preferred_element_type=jnp.float32)


def _matmul(x, w):
    m, _ = x.shape
    _, n = w.shape
    return pl.pallas_call(
        _matmul_body,
        out_shape=jax.ShapeDtypeStruct((m, n), jnp.float32),
    )(x, w)


def _update_body(h_ref, w_ref, p_ref, o_ref):
    o_ref[...] = (jnp.dot(h_ref[...], w_ref[...],
                          preferred_element_type=jnp.float32)
                  + p_ref[0] + p_ref[1])


def _self_plus_agg(h, w, parts):
    return pl.pallas_call(
        _update_body,
        out_shape=jax.ShapeDtypeStruct(h.shape, jnp.float32),
    )(h, w, parts)


def _tail_body(h_ref, batch_ref, p1w_ref, p1b_ref, p2w_ref, p2b_ref,
               p3w_ref, p3b_ref, ow_ref, ob_ref, lng_ref, lnb_ref, out_ref):
    h = h_ref[...]
    a = _silu(jnp.dot(h, p1w_ref[...], preferred_element_type=jnp.float32)
              + p1b_ref[...])
    a = _silu(jnp.dot(a, p2w_ref[...], preferred_element_type=jnp.float32)
              + p2b_ref[...])
    logits = (jnp.dot(a, p3w_ref[...], preferred_element_type=jnp.float32)
              + p3b_ref[...])
    batch = batch_ref[...]
    seg_ids = jax.lax.broadcasted_iota(jnp.int32, (N, NB), 1)
    onehot = batch == seg_ids
    onehot_f = onehot.astype(jnp.float32)
    masked = jnp.where(onehot, logits, jnp.float32(-3e38))
    seg_max = jnp.max(masked, axis=0, keepdims=True)
    maxn = jnp.sum(onehot_f * seg_max, axis=1, keepdims=True)
    zexp = jnp.exp(logits - maxn)
    denom = jnp.sum(onehot_f * zexp, axis=0, keepdims=True)
    wgt = onehot_f * zexp
    g = jax.lax.dot_general(wgt, h, (((0,), (0,)), ((), ())),
                            preferred_element_type=jnp.float32)
    g = g / denom.reshape(NB, 1)
    o = (jnp.dot(g, ow_ref[...], preferred_element_type=jnp.float32)
         + ob_ref[...])
    mu = jnp.mean(o, axis=1, keepdims=True)
    var = jnp.mean(o * o, axis=1, keepdims=True) - mu * mu
    out_ref[...] = ((o - mu) * jax.lax.rsqrt(var + 1e-5)
                    * lng_ref[...] + lnb_ref[...])


def _tail(h, batch, params):
    return pl.pallas_call(
        _tail_body,
        out_shape=jax.ShapeDtypeStruct((NB, OUT_DIM), jnp.float32),
    )(h, batch.reshape(N, 1),
      params['p1_w'], params['p1_b'].reshape(1, POOL_H),
      params['p2_w'], params['p2_b'].reshape(1, POOL_H),
      params['p3_w'], params['p3_b'].reshape(1, 1),
      params['o_w'], params['o_b'].reshape(1, OUT_DIM),
      params['ln_g'].reshape(1, OUT_DIM), params['ln_b'].reshape(1, OUT_DIM))


def kernel(x, pos, edge_index, batch, params):
    src = edge_index[0]
    dst = edge_index[1]
    h = _matmul(x, params['W_in'] / np.sqrt(D_IN))
    pos_pad = jnp.pad(pos, ((0, 0), (0, MUL - 3)))
    possrc = _sc_gather(pos_pad, src)
    posdst = _sc_gather(pos_pad, dst)
    for lp in params['layers']:
        hsrc = _sc_gather(h, src)
        msg = _edge_msg(possrc, posdst, hsrc,
                        lp['w1'], lp['b1'], lp['w2'], lp['b2'],
                        lp['w3'], lp['b3'])
        parts = _sc_scatter(msg.reshape(E, MUL), dst)
        h = _self_plus_agg(h, lp['W_si'] / np.sqrt(MUL), parts)
    return _tail(h, batch, params)


# speedup vs baseline: 10.8997x; 1.2278x over previous
import functools

import numpy as np
import jax
import jax.numpy as jnp
from jax import lax
from jax.experimental import pallas as pl
from jax.experimental.pallas import tpu as pltpu
from jax.experimental.pallas import tpu_sc as plsc

N = 10000
E = 160000
D_IN = 128
MUL = 16
RB = 8
RH = 32
POOL_H = 128
OUT_DIM = 512
NB = 16

TE = 6400
NW = 32
EPW = E // NW
NPT = N // 16

def _sc_mesh():
    return plsc.VectorSubcoreMesh(core_axis_name="c", subcore_axis_name="s")


_SC_PARAMS = pltpu.CompilerParams(use_tc_tiling_on_sc=False)
_RBF_NORM = np.float32(1.0 / (np.sqrt(2.0 * 3.14159) * 0.5))


def _silu(v):
    return v * jax.nn.sigmoid(v)


def _sc_gather(table, idx):
    @functools.partial(
        pl.kernel,
        out_type=jax.ShapeDtypeStruct((E, MUL), jnp.float32),
        mesh=_sc_mesh(),
        compiler_params=_SC_PARAMS,
        scratch_types=[
            pltpu.VMEM((EPW,), jnp.int32),
            pltpu.VMEM((EPW, MUL), jnp.float32),
            pltpu.SemaphoreType.DMA,
        ],
    )
    def k(table_hbm, idx_hbm, out_hbm, idx_v, rows_v, sem):
        wid = lax.axis_index("s") * 2 + lax.axis_index("c")
        base = wid * EPW
        pltpu.sync_copy(idx_hbm.at[pl.ds(base, EPW)], idx_v)
        pltpu.async_copy(table_hbm.at[idx_v], rows_v, sem).wait()
        pltpu.sync_copy(rows_v, out_hbm.at[pl.ds(base, EPW)])

    return k(table, idx)


def _sc_scatter(msg, idx):
    @functools.partial(
        pl.kernel,
        out_type=jax.ShapeDtypeStruct((2, N, MUL), jnp.float32),
        mesh=_sc_mesh(),
        compiler_params=_SC_PARAMS,
        scratch_types=[
            pltpu.VMEM((EPW,), jnp.int32),
            pltpu.VMEM((EPW, MUL), jnp.float32),
            pltpu.VMEM((NPT, MUL), jnp.float32),
            pltpu.VMEM_SHARED((N, MUL), jnp.float32),
            pltpu.SemaphoreType.DMA,
        ],
    )
    def k(msg_hbm, idx_hbm, out_hbm, idx_v, msg_v, node_v, acc_sh, sem):
        c = lax.axis_index("c")
        s = lax.axis_index("s")
        wid = s * 2 + c

        def zrow(i, _):
            node_v[i, :] = jnp.zeros((MUL,), jnp.float32)
            return 0
        lax.fori_loop(0, NPT, zrow, 0)
        pltpu.sync_copy(node_v, acc_sh.at[pl.ds(s * NPT, NPT)])
        plsc.subcore_barrier()

        base = wid * EPW
        pltpu.sync_copy(idx_hbm.at[pl.ds(base, EPW)], idx_v)
        pltpu.sync_copy(msg_hbm.at[pl.ds(base, EPW)], msg_v)
        pltpu.sync_copy(msg_v, acc_sh.at[idx_v], add=True)
        plsc.subcore_barrier()

        pltpu.sync_copy(acc_sh.at[pl.ds(s * NPT, NPT)], node_v)
        pltpu.sync_copy(node_v, out_hbm.at[c, pl.ds(s * NPT, NPT)])

    return k(msg, idx)


_Q = np.zeros((MUL, MUL * MUL), np.float32)
for _u in range(MUL):
    _Q[_u, _u * MUL:(_u + 1) * MUL] = 1.0
_R = np.zeros((MUL * MUL, MUL), np.float32)
for _u in range(MUL):
    for _v in range(MUL):
        _R[_u * MUL + _v, _v] = 1.0


_S = np.zeros((128, 128), np.float32)
for _j in range(8):
    for _k in range(3):
        for _i in range(RB):
            _S[16 * _j + _k, 16 * _j + _i] = 1.0


def _edge_body(ps_ref, pd_ref, hsrc_ref, w1bd_ref, b1bd_ref, w2bd_ref,
               b2bd_ref, w3_ref, b3_ref, q_ref, r_ref, s_ref, msg_ref):
    ps8 = ps_ref[...]
    pd8 = pd_ref[...]
    hs8 = hsrc_ref[...]
    diff = pd8 - ps8
    d2 = jnp.dot(diff * diff, s_ref[...],
                 preferred_element_type=jnp.float32)
    dist = jnp.maximum(jnp.sqrt(d2), 1e-6)
    li = jax.lax.broadcasted_iota(jnp.int32, (1, 128), 1) % 16
    centers = jnp.where(li < RB, li.astype(jnp.float32)
                        * np.float32(6.0 / (RB - 1)), jnp.float32(1e5))
    rbf = jnp.exp(-((dist - centers) ** 2) * 2.0) * _RBF_NORM
    h1 = _silu(jnp.dot(rbf, w1bd_ref[...],
                       preferred_element_type=jnp.float32) + b1bd_ref[...])
    h2 = _silu(jnp.dot(h1, w2bd_ref[...],
                       preferred_element_type=jnp.float32) + b2bd_ref[...])
    w3 = w3_ref[...]
    b3 = b3_ref[...]
    q = q_ref[...]
    r = r_ref[...]
    msgs = []
    for j in range(8):
        twj = (jnp.dot(h2[:, 32 * j:32 * (j + 1)], w3,
                       preferred_element_type=jnp.float32) + b3)
        hbj = jnp.dot(hs8[:, 16 * j:16 * (j + 1)], q,
                      preferred_element_type=jnp.float32)
        msgs.append(jnp.dot(twj * hbj, r,
                            preferred_element_type=jnp.float32))
    msg_ref[...] = (jnp.concatenate(msgs, axis=1)
                    * np.float32(1.0 / np.sqrt(MUL)))


def _edge_msg(possrc, posdst, hsrc, w1, b1, w2, b2, w3, b3):
    grid = (E // TE,)
    pk = lambda a: a.reshape(E // 8, 128)
    eye8 = jnp.eye(8, dtype=jnp.float32)
    w1bd = jnp.kron(eye8, jnp.pad(w1, ((0, MUL - RB), (0, 0))))
    w2bd = jnp.kron(eye8, w2)
    b1bd = jnp.tile(b1, 8).reshape(1, 8 * RH)
    b2bd = jnp.tile(b2, 8).reshape(1, 8 * RH)
    return pl.pallas_call(
        _edge_body,
        grid=grid,
        in_specs=[
            pl.BlockSpec((TE // 8, 128), lambda i: (i, 0)),
            pl.BlockSpec((TE // 8, 128), lambda i: (i, 0)),
            pl.BlockSpec((TE // 8, 128), lambda i: (i, 0)),
            pl.BlockSpec((128, 8 * RH), lambda i: (0, 0)),
            pl.BlockSpec((1, 8 * RH), lambda i: (0, 0)),
            pl.BlockSpec((8 * RH, 8 * RH), lambda i: (0, 0)),
            pl.BlockSpec((1, 8 * RH), lambda i: (0, 0)),
            pl.BlockSpec((RH, MUL * MUL), lambda i: (0, 0)),
            pl.BlockSpec((1, MUL * MUL), lambda i: (0, 0)),
            pl.BlockSpec((MUL, MUL * MUL), lambda i: (0, 0)),
            pl.BlockSpec((MUL * MUL, MUL), lambda i: (0, 0)),
            pl.BlockSpec((128, 128), lambda i: (0, 0)),
        ],
        out_specs=pl.BlockSpec((TE // 8, 128), lambda i: (i, 0)),
        out_shape=jax.ShapeDtypeStruct((E // 8, 128), jnp.float32),
    )(pk(possrc), pk(posdst), pk(hsrc),
      w1bd, b1bd, w2bd, b2bd,
      w3, b3.reshape(1, MUL * MUL), jnp.asarray(_Q), jnp.asarray(_R),
      jnp.asarray(_S))


def _matmul_body(x_ref, w_ref, o_ref):
    o_ref[...] = jnp.dot(x_ref[...], w_ref[...],
                         preferred_element_type=jnp.float32)


def _matmul(x, w):
    m, _ = x.shape
    _, n = w.shape
    return pl.pallas_call(
        _matmul_body,
        out_shape=jax.ShapeDtypeStruct((m, n), jnp.float32),
    )(x, w)


def _update_body(h_ref, w_ref, p_ref, o_ref):
    o_ref[...] = (jnp.dot(h_ref[...], w_ref[...],
                          preferred_element_type=jnp.float32)
                  + p_ref[0] + p_ref[1])


def _self_plus_agg(h, w, parts):
    return pl.pallas_call(
        _update_body,
        out_shape=jax.ShapeDtypeStruct(h.shape, jnp.float32),
    )(h, w, parts)


def _tail_body(h_ref, batch_ref, p1w_ref, p1b_ref, p2w_ref, p2b_ref,
               p3w_ref, p3b_ref, ow_ref, ob_ref, lng_ref, lnb_ref, out_ref):
    h = h_ref[...]
    a = _silu(jnp.dot(h, p1w_ref[...], preferred_element_type=jnp.float32)
              + p1b_ref[...])
    a = _silu(jnp.dot(a, p2w_ref[...], preferred_element_type=jnp.float32)
              + p2b_ref[...])
    logits = (jnp.dot(a, p3w_ref[...], preferred_element_type=jnp.float32)
              + p3b_ref[...])
    batch = batch_ref[...]
    seg_ids = jax.lax.broadcasted_iota(jnp.int32, (N, NB), 1)
    onehot = batch == seg_ids
    onehot_f = onehot.astype(jnp.float32)
    masked = jnp.where(onehot, logits, jnp.float32(-3e38))
    seg_max = jnp.max(masked, axis=0, keepdims=True)
    maxn = jnp.sum(onehot_f * seg_max, axis=1, keepdims=True)
    zexp = jnp.exp(logits - maxn)
    denom = jnp.sum(onehot_f * zexp, axis=0, keepdims=True)
    wgt = onehot_f * zexp
    g = jax.lax.dot_general(wgt, h, (((0,), (0,)), ((), ())),
                            preferred_element_type=jnp.float32)
    g = g / denom.reshape(NB, 1)
    o = (jnp.dot(g, ow_ref[...], preferred_element_type=jnp.float32)
         + ob_ref[...])
    mu = jnp.mean(o, axis=1, keepdims=True)
    var = jnp.mean(o * o, axis=1, keepdims=True) - mu * mu
    out_ref[...] = ((o - mu) * jax.lax.rsqrt(var + 1e-5)
                    * lng_ref[...] + lnb_ref[...])


def _tail(h, batch, params):
    return pl.pallas_call(
        _tail_body,
        out_shape=jax.ShapeDtypeStruct((NB, OUT_DIM), jnp.float32),
    )(h, batch.reshape(N, 1),
      params['p1_w'], params['p1_b'].reshape(1, POOL_H),
      params['p2_w'], params['p2_b'].reshape(1, POOL_H),
      params['p3_w'], params['p3_b'].reshape(1, 1),
      params['o_w'], params['o_b'].reshape(1, OUT_DIM),
      params['ln_g'].reshape(1, OUT_DIM), params['ln_b'].reshape(1, OUT_DIM))


def kernel(x, pos, edge_index, batch, params):
    src = edge_index[0]
    dst = edge_index[1]
    h = _matmul(x, params['W_in'] / np.sqrt(D_IN))
    pos_pad = jnp.pad(pos, ((0, 0), (0, MUL - 3)))
    possrc = _sc_gather(pos_pad, src)
    posdst = _sc_gather(pos_pad, dst)
    for lp in params['layers']:
        hsrc = _sc_gather(h, src)
        msg = _edge_msg(possrc, posdst, hsrc,
                        lp['w1'], lp['b1'], lp['w2'], lp['b2'],
                        lp['w3'], lp['b3'])
        parts = _sc_scatter(msg.reshape(E, MUL), dst)
        h = _self_plus_agg(h, lp['W_si'] / np.sqrt(MUL), parts)
    return _tail(h, batch, params)


# speedup vs baseline: 11.1389x; 1.0220x over previous
import functools

import numpy as np
import jax
import jax.numpy as jnp
from jax import lax
from jax.experimental import pallas as pl
from jax.experimental.pallas import tpu as pltpu
from jax.experimental.pallas import tpu_sc as plsc

N = 10000
E = 160000
D_IN = 128
MUL = 16
RB = 8
RH = 32
POOL_H = 128
OUT_DIM = 512
NB = 16

TE = 8000
NW = 32
EPW = E // NW
NPT = N // 16

def _sc_mesh():
    return plsc.VectorSubcoreMesh(core_axis_name="c", subcore_axis_name="s")


_SC_PARAMS = pltpu.CompilerParams(use_tc_tiling_on_sc=False)
_RBF_NORM = np.float32(1.0 / (np.sqrt(2.0 * 3.14159) * 0.5))


def _silu(v):
    return v * jax.nn.sigmoid(v)


def _sc_gather(table, idx):
    @functools.partial(
        pl.kernel,
        out_type=jax.ShapeDtypeStruct((E, MUL), jnp.float32),
        mesh=_sc_mesh(),
        compiler_params=_SC_PARAMS,
        scratch_types=[
            pltpu.VMEM((EPW,), jnp.int32),
            pltpu.VMEM((EPW, MUL), jnp.float32),
            pltpu.SemaphoreType.DMA,
        ],
    )
    def k(table_hbm, idx_hbm, out_hbm, idx_v, rows_v, sem):
        wid = lax.axis_index("s") * 2 + lax.axis_index("c")
        base = wid * EPW
        pltpu.sync_copy(idx_hbm.at[pl.ds(base, EPW)], idx_v)
        pltpu.async_copy(table_hbm.at[idx_v], rows_v, sem).wait()
        pltpu.sync_copy(rows_v, out_hbm.at[pl.ds(base, EPW)])

    return k(table, idx)


def _sc_scatter(msg, idx):
    @functools.partial(
        pl.kernel,
        out_type=jax.ShapeDtypeStruct((2, N, MUL), jnp.float32),
        mesh=_sc_mesh(),
        compiler_params=_SC_PARAMS,
        scratch_types=[
            pltpu.VMEM((EPW,), jnp.int32),
            pltpu.VMEM((EPW, MUL), jnp.float32),
            pltpu.VMEM((NPT, MUL), jnp.float32),
            pltpu.VMEM_SHARED((N, MUL), jnp.float32),
            pltpu.SemaphoreType.DMA,
        ],
    )
    def k(msg_hbm, idx_hbm, out_hbm, idx_v, msg_v, node_v, acc_sh, sem):
        c = lax.axis_index("c")
        s = lax.axis_index("s")
        wid = s * 2 + c

        def zrow(i, _):
            node_v[i, :] = jnp.zeros((MUL,), jnp.float32)
            return 0
        lax.fori_loop(0, NPT, zrow, 0)
        pltpu.sync_copy(node_v, acc_sh.at[pl.ds(s * NPT, NPT)])
        plsc.subcore_barrier()

        base = wid * EPW
        pltpu.sync_copy(idx_hbm.at[pl.ds(base, EPW)], idx_v)
        pltpu.sync_copy(msg_hbm.at[pl.ds(base, EPW)], msg_v)
        pltpu.sync_copy(msg_v, acc_sh.at[idx_v], add=True)
        plsc.subcore_barrier()

        pltpu.sync_copy(acc_sh.at[pl.ds(s * NPT, NPT)], node_v)
        pltpu.sync_copy(node_v, out_hbm.at[c, pl.ds(s * NPT, NPT)])

    return k(msg, idx)


_Q = np.zeros((MUL, MUL * MUL), np.float32)
for _u in range(MUL):
    _Q[_u, _u * MUL:(_u + 1) * MUL] = 1.0
_R = np.zeros((MUL * MUL, MUL), np.float32)
for _u in range(MUL):
    for _v in range(MUL):
        _R[_u * MUL + _v, _v] = 1.0


_S = np.zeros((128, 128), np.float32)
for _j in range(8):
    for _k in range(3):
        for _i in range(RB):
            _S[16 * _j + _k, 16 * _j + _i] = 1.0


def _edge_body(ps_ref, pd_ref, hsrc_ref, w1bd_ref, b1bd_ref, w2bd_ref,
               b2bd_ref, w3_ref, b3_ref, q_ref, r_ref, s_ref, msg_ref):
    ps8 = ps_ref[...]
    pd8 = pd_ref[...]
    hs8 = hsrc_ref[...]
    diff = pd8 - ps8
    d2 = jnp.dot(diff * diff, s_ref[...],
                 preferred_element_type=jnp.float32)
    dist = jnp.maximum(jnp.sqrt(d2), 1e-6)
    li = jax.lax.broadcasted_iota(jnp.int32, (1, 128), 1) % 16
    centers = jnp.where(li < RB, li.astype(jnp.float32)
                        * np.float32(6.0 / (RB - 1)), jnp.float32(1e5))
    rbf = jnp.exp(-((dist - centers) ** 2) * 2.0) * _RBF_NORM
    h1 = _silu(jnp.dot(rbf, w1bd_ref[...],
                       preferred_element_type=jnp.float32) + b1bd_ref[...])
    h2 = _silu(jnp.dot(h1, w2bd_ref[...],
                       preferred_element_type=jnp.float32) + b2bd_ref[...])
    w3 = w3_ref[...]
    b3 = b3_ref[...]
    q = q_ref[...]
    r = r_ref[...]
    msgs = []
    for j in range(8):
        twj = (jnp.dot(h2[:, 32 * j:32 * (j + 1)], w3,
                       preferred_element_type=jnp.float32) + b3)
        hbj = jnp.dot(hs8[:, 16 * j:16 * (j + 1)], q,
                      preferred_element_type=jnp.float32)
        msgs.append(jnp.dot(twj * hbj, r,
                            preferred_element_type=jnp.float32))
    msg_ref[...] = (jnp.concatenate(msgs, axis=1)
                    * np.float32(1.0 / np.sqrt(MUL)))


def _edge_msg(possrc, posdst, hsrc, w1, b1, w2, b2, w3, b3):
    grid = (E // TE,)
    pk = lambda a: a.reshape(E // 8, 128)
    eye8 = jnp.eye(8, dtype=jnp.float32)
    w1bd = jnp.kron(eye8, jnp.pad(w1, ((0, MUL - RB), (0, 0))))
    w2bd = jnp.kron(eye8, w2)
    b1bd = jnp.tile(b1, 8).reshape(1, 8 * RH)
    b2bd = jnp.tile(b2, 8).reshape(1, 8 * RH)
    return pl.pallas_call(
        _edge_body,
        grid=grid,
        in_specs=[
            pl.BlockSpec((TE // 8, 128), lambda i: (i, 0)),
            pl.BlockSpec((TE // 8, 128), lambda i: (i, 0)),
            pl.BlockSpec((TE // 8, 128), lambda i: (i, 0)),
            pl.BlockSpec((128, 8 * RH), lambda i: (0, 0)),
            pl.BlockSpec((1, 8 * RH), lambda i: (0, 0)),
            pl.BlockSpec((8 * RH, 8 * RH), lambda i: (0, 0)),
            pl.BlockSpec((1, 8 * RH), lambda i: (0, 0)),
            pl.BlockSpec((RH, MUL * MUL), lambda i: (0, 0)),
            pl.BlockSpec((1, MUL * MUL), lambda i: (0, 0)),
            pl.BlockSpec((MUL, MUL * MUL), lambda i: (0, 0)),
            pl.BlockSpec((MUL * MUL, MUL), lambda i: (0, 0)),
            pl.BlockSpec((128, 128), lambda i: (0, 0)),
        ],
        out_specs=pl.BlockSpec((TE // 8, 128), lambda i: (i, 0)),
        out_shape=jax.ShapeDtypeStruct((E // 8, 128), jnp.float32),
    )(pk(possrc), pk(posdst), pk(hsrc),
      w1bd, b1bd, w2bd, b2bd,
      w3, b3.reshape(1, MUL * MUL), jnp.asarray(_Q), jnp.asarray(_R),
      jnp.asarray(_S))


def _matmul_body(x_ref, w_ref, o_ref):
    o_ref[...] = jnp.dot(x_ref[...], w_ref[...],
                         preferred_element_type=jnp.float32)


def _matmul(x, w):
    m, _ = x.shape
    _, n = w.shape
    return pl.pallas_call(
        _matmul_body,
        out_shape=jax.ShapeDtypeStruct((m, n), jnp.float32),
    )(x, w)


def _update_body(h_ref, w_ref, p_ref, o_ref):
    o_ref[...] = (jnp.dot(h_ref[...], w_ref[...],
                          preferred_element_type=jnp.float32)
                  + p_ref[0] + p_ref[1])


def _self_plus_agg(h, w, parts):
    return pl.pallas_call(
        _update_body,
        out_shape=jax.ShapeDtypeStruct(h.shape, jnp.float32),
    )(h, w, parts)


def _tail_body(h_ref, batch_ref, p1w_ref, p1b_ref, p2w_ref, p2b_ref,
               p3w_ref, p3b_ref, ow_ref, ob_ref, lng_ref, lnb_ref, out_ref):
    h = h_ref[...]
    a = _silu(jnp.dot(h, p1w_ref[...], preferred_element_type=jnp.float32)
              + p1b_ref[...])
    a = _silu(jnp.dot(a, p2w_ref[...], preferred_element_type=jnp.float32)
              + p2b_ref[...])
    logits = (jnp.dot(a, p3w_ref[...], preferred_element_type=jnp.float32)
              + p3b_ref[...])
    batch = batch_ref[...]
    seg_ids = jax.lax.broadcasted_iota(jnp.int32, (N, NB), 1)
    onehot = batch == seg_ids
    onehot_f = onehot.astype(jnp.float32)
    masked = jnp.where(onehot, logits, jnp.float32(-3e38))
    seg_max = jnp.max(masked, axis=0, keepdims=True)
    maxn = jnp.sum(onehot_f * seg_max, axis=1, keepdims=True)
    zexp = jnp.exp(logits - maxn)
    denom = jnp.sum(onehot_f * zexp, axis=0, keepdims=True)
    wgt = onehot_f * zexp
    g = jax.lax.dot_general(wgt, h, (((0,), (0,)), ((), ())),
                            preferred_element_type=jnp.float32)
    g = g / denom.reshape(NB, 1)
    o = (jnp.dot(g, ow_ref[...], preferred_element_type=jnp.float32)
         + ob_ref[...])
    mu = jnp.mean(o, axis=1, keepdims=True)
    var = jnp.mean(o * o, axis=1, keepdims=True) - mu * mu
    out_ref[...] = ((o - mu) * jax.lax.rsqrt(var + 1e-5)
                    * lng_ref[...] + lnb_ref[...])


def _tail(h, batch, params):
    return pl.pallas_call(
        _tail_body,
        out_shape=jax.ShapeDtypeStruct((NB, OUT_DIM), jnp.float32),
    )(h, batch.reshape(N, 1),
      params['p1_w'], params['p1_b'].reshape(1, POOL_H),
      params['p2_w'], params['p2_b'].reshape(1, POOL_H),
      params['p3_w'], params['p3_b'].reshape(1, 1),
      params['o_w'], params['o_b'].reshape(1, OUT_DIM),
      params['ln_g'].reshape(1, OUT_DIM), params['ln_b'].reshape(1, OUT_DIM))


def kernel(x, pos, edge_index, batch, params):
    src = edge_index[0]
    dst = edge_index[1]
    h = _matmul(x, params['W_in'] / np.sqrt(D_IN))
    pos_pad = jnp.pad(pos, ((0, 0), (0, MUL - 3)))
    possrc = _sc_gather(pos_pad, src)
    posdst = _sc_gather(pos_pad, dst)
    for lp in params['layers']:
        hsrc = _sc_gather(h, src)
        msg = _edge_msg(possrc, posdst, hsrc,
                        lp['w1'], lp['b1'], lp['w2'], lp['b2'],
                        lp['w3'], lp['b3'])
        parts = _sc_scatter(msg.reshape(E, MUL), dst)
        h = _self_plus_agg(h, lp['W_si'] / np.sqrt(MUL), parts)
    return _tail(h, batch, params)
